# baseline (speedup 1.0000x reference)
import functools

import jax
import jax.numpy as jnp
from jax import lax
from jax.experimental import pallas as pl
from jax.experimental.pallas import tpu as pltpu
from jax.experimental.pallas import tpu_sc as plsc

NC = 2
NS = 16
NW = NC * NS
L = 16

BATCH = 16384
HIST = 20
DIM = 64
V = 1000000
VPAIR = V // 2
BW = BATCH // NW
B_PER_W = BW * HIST
CHUNK = 128
NBUF = 2
NSTEPS = B_PER_W // CHUNK
NROUNDS = NSTEPS // NBUF
SW = 129


@functools.partial(
    pl.kernel,
    out_type=jax.ShapeDtypeStruct((HIST, DIM, BATCH), jnp.float32),
    mesh=plsc.VectorSubcoreMesh(core_axis_name="c", subcore_axis_name="s"),
    scratch_types=[
        pltpu.VMEM((HIST, BW), jnp.int32),
        pltpu.VMEM((B_PER_W,), jnp.int32),
        pltpu.VMEM((B_PER_W,), jnp.int32),
        pltpu.VMEM((NBUF, CHUNK, 2 * DIM), jnp.float32),
        pltpu.VMEM((2, DIM, SW), jnp.float32),
        pltpu.SemaphoreType.DMA((NBUF,)),
        pltpu.SemaphoreType.DMA((2,)),
    ],
    compiler_params=pltpu.CompilerParams(
        use_tc_tiling_on_sc=True, needs_layout_passes=False),
)
def _embed_kernel(w2_hbm, it_hbm, out_hbm, idx_v, gidx_v, off_v, pairs_v,
                  stage_v, gsem, ssem):
    sid = lax.axis_index("s")
    cid = lax.axis_index("c")
    wid = sid * NC + cid
    b0 = wid * BW

    iota = lax.iota(jnp.int32, L)

    pltpu.sync_copy(it_hbm.at[:, pl.ds(b0, BW)], idx_v)

    @pl.loop(0, HIST)
    def _h(h):
        @pl.loop(0, BW // L)
        def _g(g):
            v = idx_v[h, pl.ds(g * L, L)]
            gidx_v[pl.ds(h * BW + g * L, L)] = v >> 1
            off_v[pl.ds(h * BW + g * L, L)] = (v & 1) << 6

    def start_gather(k, bb):
        pltpu.async_copy(
            w2_hbm.at[gidx_v.at[pl.ds(k * CHUNK, CHUNK)]],
            pairs_v.at[bb], gsem.at[bb],
        )

    def extract(k, bb, sb):
        @pl.loop(0, CHUNK // L)
        def _jg(jg):
            rows = jg * L + iota
            offv = off_v[pl.ds(k * CHUNK + jg * L, L)]

            @pl.loop(0, DIM)
            def _d(d):
                val = plsc.load_gather(pairs_v.at[bb], [rows, offv + d])
                stage_v[sb, d, pl.ds(jg * L, L)] = val

    for bb in range(NBUF):
        start_gather(bb, bb)

    @pl.loop(0, NROUNDS)
    def _round(g):
        for bb in range(NBUF):
            k = g * NBUF + bb
            sb = bb
            pltpu.make_async_copy(
                w2_hbm.at[gidx_v.at[pl.ds(0, CHUNK)]], pairs_v.at[bb],
                gsem.at[bb],
            ).wait()
            @pl.when(k >= 2)
            def _():
                pltpu.make_async_copy(
                    stage_v.at[sb, :, pl.ds(0, 128)],
                    out_hbm.at[0, :, pl.ds(0, 128)], ssem.at[sb],
                ).wait()
            extract(k, bb, sb)
            h = k // 4
            bstart = b0 + 128 * (k % 4)
            pltpu.async_copy(
                stage_v.at[sb, :, pl.ds(0, 128)],
                out_hbm.at[h, :, pl.ds(bstart, 128)], ssem.at[sb],
            )
            @pl.when(k + NBUF < NSTEPS)
            def _():
                start_gather(k + NBUF, bb)

    for sb in range(2):
        pltpu.make_async_copy(
            stage_v.at[sb, :, pl.ds(0, 128)],
            out_hbm.at[0, :, pl.ds(0, 128)], ssem.at[sb],
        ).wait()


def kernel(input_, weight):
    it = input_.T.astype(jnp.int32)
    w2 = jnp.reshape(weight, (VPAIR, 2 * DIM))
    out_t = _embed_kernel(w2, it)
    return jnp.transpose(out_t, (2, 0, 1))


# speedup vs baseline: 1.0860x; 1.0860x over previous
import functools

import jax
import jax.numpy as jnp
from jax import lax
from jax.experimental import pallas as pl
from jax.experimental.pallas import tpu as pltpu
from jax.experimental.pallas import tpu_sc as plsc

NC = 2
NS = 16
NW = NC * NS
L = 16

BATCH = 16384
HIST = 20
DIM = 64
V = 1000000
VPAIR = V // 2
BW = BATCH // NW
B_PER_W = BW * HIST
CHUNK = 128
NBUF = 2
NSTEPS = B_PER_W // CHUNK
NROUNDS = NSTEPS // NBUF
SW = 129


@functools.partial(
    pl.kernel,
    out_type=jax.ShapeDtypeStruct((HIST, DIM, BATCH), jnp.float32),
    mesh=plsc.VectorSubcoreMesh(core_axis_name="c", subcore_axis_name="s"),
    scratch_types=[
        pltpu.VMEM((HIST, BW), jnp.int32),
        pltpu.VMEM((B_PER_W,), jnp.int32),
        pltpu.VMEM((B_PER_W,), jnp.int32),
        pltpu.VMEM((NBUF, CHUNK, 2 * DIM), jnp.float32),
        pltpu.VMEM((2, DIM, SW), jnp.float32),
        pltpu.SemaphoreType.DMA((NBUF,)),
        pltpu.SemaphoreType.DMA((2,)),
    ],
    compiler_params=pltpu.CompilerParams(
        use_tc_tiling_on_sc=True, needs_layout_passes=False),
)
def _embed_kernel(w2_hbm, it_hbm, out_hbm, idx_v, gidx_v, off_v, pairs_v,
                  stage_v, gsem, ssem):
    sid = lax.axis_index("s")
    cid = lax.axis_index("c")
    wid = sid * NC + cid
    b0 = wid * BW

    iota = lax.iota(jnp.int32, L)

    pltpu.sync_copy(it_hbm.at[:, pl.ds(b0, BW)], idx_v)

    @pl.loop(0, HIST)
    def _h(h):
        @pl.loop(0, BW // L)
        def _g(g):
            v = idx_v[h, pl.ds(g * L, L)]
            gidx_v[pl.ds(h * BW + g * L, L)] = v >> 1
            off_v[pl.ds(h * BW + g * L, L)] = (v & 1) << 6

    def start_gather(k, bb):
        pltpu.async_copy(
            w2_hbm.at[gidx_v.at[pl.ds(k * CHUNK, CHUNK)]],
            pairs_v.at[bb], gsem.at[bb],
        )

    rsc = [16 * c + iota for c in range(4)]

    def extract(k, bb, sb):
        @pl.loop(0, CHUNK // L)
        def _jg(jg):
            offv = off_v[pl.ds(k * CHUNK + jg * L, L)]
            for j in range(L):
                colv = jnp.full((L,), jg * L + j, dtype=jnp.int32)
                off_j = offv[j]
                for c in range(4):
                    val = pairs_v[bb, jg * L + j, pl.ds(off_j + 16 * c, L)]
                    plsc.store_scatter(stage_v.at[sb], [rsc[c], colv], val)

    for bb in range(NBUF):
        start_gather(bb, bb)

    @pl.loop(0, NROUNDS)
    def _round(g):
        for bb in range(NBUF):
            k = g * NBUF + bb
            sb = bb
            pltpu.make_async_copy(
                w2_hbm.at[gidx_v.at[pl.ds(0, CHUNK)]], pairs_v.at[bb],
                gsem.at[bb],
            ).wait()
            @pl.when(k >= 2)
            def _():
                pltpu.make_async_copy(
                    stage_v.at[sb, :, pl.ds(0, 128)],
                    out_hbm.at[0, :, pl.ds(0, 128)], ssem.at[sb],
                ).wait()
            extract(k, bb, sb)
            h = k // 4
            bstart = b0 + 128 * (k % 4)
            pltpu.async_copy(
                stage_v.at[sb, :, pl.ds(0, 128)],
                out_hbm.at[h, :, pl.ds(bstart, 128)], ssem.at[sb],
            )
            @pl.when(k + NBUF < NSTEPS)
            def _():
                start_gather(k + NBUF, bb)

    for sb in range(2):
        pltpu.make_async_copy(
            stage_v.at[sb, :, pl.ds(0, 128)],
            out_hbm.at[0, :, pl.ds(0, 128)], ssem.at[sb],
        ).wait()


def kernel(input_, weight):
    it = input_.T.astype(jnp.int32)
    w2 = jnp.reshape(weight, (VPAIR, 2 * DIM))
    out_t = _embed_kernel(w2, it)
    return jnp.transpose(out_t, (2, 0, 1))
